# Initial kernel scaffold; baseline (speedup 1.0000x reference)
import functools

import jax
import jax.numpy as jnp
from jax import lax
from jax.experimental import pallas as pl
from jax.experimental.pallas import tpu as pltpu
from jax.experimental.pallas import tpu_sc as plsc

f32 = jnp.float32

N = 100000
NP = 100096
E = 1600000
EP = 1605632
CH = EP // 128
NW = 32
CPT = CH // NW
CPS = CH // 16
GK = 4
SK = 8
ROWS_G = GK * 128
ROWS_S = SK * 128
SLAB = NP // 16
ZR = 782

_mesh = plsc.VectorSubcoreMesh(core_axis_name="c", subcore_axis_name="s")


def _gather_body(a_hbm, b_hbm, dstg, srcg, h1,
                 idxd0, idxs0, idxd1, idxs1,
                 bufa0, bufb0, bufa1, bufb1, sem0, sem1):
    cid = lax.axis_index("c")
    sid = lax.axis_index("s")
    wid = sid * 2 + cid
    base = wid * CPT
    idxd = (idxd0, idxd1)
    idxs = (idxs0, idxs1)
    bufa = (bufa0, bufa1)
    bufb = (bufb0, bufb1)
    sem = (sem0, sem1)

    def fire(h, b):
        c0 = base + h * GK
        pltpu.sync_copy(dstg.at[pl.ds(c0, GK)], idxd[b])
        pltpu.sync_copy(srcg.at[pl.ds(c0, GK)], idxs[b])
        for j in range(GK):
            sl = pl.ds(j * 128, 128)
            pltpu.async_copy(a_hbm.at[idxd[b].at[j]], bufa[b].at[sl], sem[b])
            pltpu.async_copy(b_hbm.at[idxs[b].at[j]], bufb[b].at[sl], sem[b])

    def drain(b):
        for j in range(GK):
            sl = pl.ds(j * 128, 128)
            pltpu.make_async_copy(a_hbm.at[idxd[b].at[j]], bufa[b].at[sl], sem[b]).wait()
            pltpu.make_async_copy(b_hbm.at[idxs[b].at[j]], bufb[b].at[sl], sem[b]).wait()

    fire(0, 0)
    fire(1, 1)
    n_groups = CPT // GK

    def outer(g, carry):
        for b in range(2):
            h = g * 2 + b
            drain(b)

            def add_body(i, c):
                for r in range(4):
                    row = i * 4 + r
                    for half in range(2):
                        sl = pl.ds(half * 16, 16)
                        bufa[b][row, sl] = bufa[b][row, sl] + bufb[b][row, sl]
                return c

            lax.fori_loop(0, ROWS_G // 4, add_body, 0)
            c0 = base + h * GK
            pltpu.sync_copy(bufa[b], h1.at[pl.ds(c0 * 128, ROWS_G)])

            @pl.when(h + 2 < n_groups)
            def _():
                fire(h + 2, b)
        return carry

    lax.fori_loop(0, n_groups // 2, outer, 0)


_gather_call = functools.partial(
    pl.kernel,
    out_type=jax.ShapeDtypeStruct((EP, 32), f32),
    mesh=_mesh,
    scratch_types=[
        pltpu.VMEM((GK, 128), jnp.int32),
        pltpu.VMEM((GK, 128), jnp.int32),
        pltpu.VMEM((GK, 128), jnp.int32),
        pltpu.VMEM((GK, 128), jnp.int32),
        pltpu.VMEM((ROWS_G, 32), f32),
        pltpu.VMEM((ROWS_G, 32), f32),
        pltpu.VMEM((ROWS_G, 32), f32),
        pltpu.VMEM((ROWS_G, 32), f32),
        pltpu.SemaphoreType.DMA,
        pltpu.SemaphoreType.DMA,
    ],
)(_gather_body)


def _zero_acc(acc, zbuf, sid):
    def zrow(i, c):
        zbuf[i, pl.ds(0, 16)] = jnp.zeros((16,), f32)
        return c

    lax.fori_loop(0, ZR, zrow, 0)
    for t in range(8):
        pltpu.sync_copy(zbuf, acc.at[pl.ds(sid * SLAB + t * ZR, ZR)])


def _scatter_fs_body(h3, dsts, out, acc, idxv, rows, zbuf, sem):
    cid = lax.axis_index("c")
    sid = lax.axis_index("s")
    _zero_acc(acc, zbuf, sid)
    plsc.subcore_barrier()
    base = sid * CPS

    def grp(g, carry):
        c0 = base + g * SK
        pltpu.sync_copy(dsts.at[pl.ds(c0, SK)], idxv)
        pltpu.sync_copy(h3.at[pl.ds(c0 * 128, ROWS_S), pl.ds(cid * 16, 16)], rows)
        for j in range(SK):
            sl = pl.ds(j * 128, 128)
            pltpu.async_copy(rows.at[sl], acc.at[idxv.at[j]], sem, add=True)
        for j in range(SK):
            sl = pl.ds(j * 128, 128)
            pltpu.make_async_copy(rows.at[sl], acc.at[idxv.at[j]], sem).wait()
        return carry

    lax.fori_loop(0, CPS // SK, grp, 0)
    plsc.subcore_barrier()
    pltpu.sync_copy(acc.at[pl.ds(sid * SLAB, SLAB)],
                    out.at[pl.ds(sid * SLAB, SLAB), pl.ds(cid * 16, 16)])


_scatter_fs_call = functools.partial(
    pl.kernel,
    out_type=jax.ShapeDtypeStruct((NP, 32), f32),
    mesh=_mesh,
    scratch_types=[
        pltpu.VMEM_SHARED((NP, 16), f32),
        pltpu.VMEM((SK, 128), jnp.int32),
        pltpu.VMEM((ROWS_S, 16), f32),
        pltpu.VMEM((ZR, 16), f32),
        pltpu.SemaphoreType.DMA,
    ],
)(_scatter_fs_body)


def _scatter_es_body(h3, dsts, out0, out1, acc, idxv, rows, zbuf, sem):
    cid = lax.axis_index("c")
    sid = lax.axis_index("s")
    _zero_acc(acc, zbuf, sid)
    plsc.subcore_barrier()
    wid = sid * 2 + cid
    base = wid * CPT

    def grp(g, carry):
        c0 = base + g * SK
        pltpu.sync_copy(dsts.at[pl.ds(c0, SK)], idxv)
        pltpu.sync_copy(h3.at[pl.ds(c0 * 128, ROWS_S)], rows)
        for j in range(SK):
            sl = pl.ds(j * 128, 128)
            pltpu.async_copy(rows.at[sl], acc.at[idxv.at[j]], sem, add=True)
        for j in range(SK):
            sl = pl.ds(j * 128, 128)
            pltpu.make_async_copy(rows.at[sl], acc.at[idxv.at[j]], sem).wait()
        return carry

    lax.fori_loop(0, CPT // SK, grp, 0)
    plsc.subcore_barrier()
    slab = pl.ds(sid * SLAB, SLAB)

    @pl.when(cid == 0)
    def _():
        pltpu.sync_copy(acc.at[slab], out0.at[slab])

    @pl.when(cid == 1)
    def _():
        pltpu.sync_copy(acc.at[slab], out1.at[slab])


_scatter_es_call = functools.partial(
    pl.kernel,
    out_type=[jax.ShapeDtypeStruct((NP, 16), f32),
              jax.ShapeDtypeStruct((NP, 16), f32)],
    mesh=_mesh,
    scratch_types=[
        pltpu.VMEM_SHARED((NP, 16), f32),
        pltpu.VMEM((SK, 128), jnp.int32),
        pltpu.VMEM((ROWS_S, 16), f32),
        pltpu.VMEM((ZR, 16), f32),
        pltpu.SemaphoreType.DMA,
    ],
)(_scatter_es_body)


def _table_tc(x_ref, wd_ref, wb_ref, bd_ref, a_ref, b_ref):
    xv = x_ref[...]
    a_ref[...] = jnp.dot(xv, wd_ref[...], preferred_element_type=f32) + bd_ref[...]
    b_ref[...] = jnp.dot(xv, wb_ref[...], preferred_element_type=f32)


def _make_table(nrows, in_cols, out_cols, grid):
    blk = nrows // grid
    return pl.pallas_call(
        _table_tc,
        grid=(grid,),
        in_specs=[
            pl.BlockSpec((blk, in_cols), lambda i: (i, 0)),
            pl.BlockSpec((in_cols, out_cols), lambda i: (0, 0)),
            pl.BlockSpec((in_cols, out_cols), lambda i: (0, 0)),
            pl.BlockSpec((1, out_cols), lambda i: (0, 0)),
        ],
        out_specs=[
            pl.BlockSpec((blk, out_cols), lambda i: (i, 0)),
            pl.BlockSpec((blk, out_cols), lambda i: (i, 0)),
        ],
        out_shape=[
            jax.ShapeDtypeStruct((nrows, out_cols), f32),
            jax.ShapeDtypeStruct((nrows, out_cols), f32),
        ],
    )


def _mid_tc(h_ref, w1_ref, b1_ref, w2_ref, b2_ref, o_ref):
    t = jnp.maximum(h_ref[...], 0.0)
    t = jnp.maximum(jnp.dot(t, w1_ref[...], preferred_element_type=f32) + b1_ref[...], 0.0)
    o_ref[...] = jnp.dot(t, w2_ref[...], preferred_element_type=f32) + b2_ref[...]


def _make_mid(out_cols):
    nrows = EP // 4
    blk = 1024
    grid = nrows // blk
    return pl.pallas_call(
        _mid_tc,
        grid=(grid,),
        in_specs=[
            pl.BlockSpec((blk, 128), lambda i: (i, 0)),
            pl.BlockSpec((128, 128), lambda i: (0, 0)),
            pl.BlockSpec((1, 128), lambda i: (0, 0)),
            pl.BlockSpec((128, out_cols), lambda i: (0, 0)),
            pl.BlockSpec((1, out_cols), lambda i: (0, 0)),
        ],
        out_specs=pl.BlockSpec((blk, out_cols), lambda i: (i, 0)),
        out_shape=jax.ShapeDtypeStruct((nrows, out_cols), f32),
    )


def _add_tc(a_ref, b_ref, o_ref):
    o_ref[...] = a_ref[...] + b_ref[...]


_add_call = pl.pallas_call(
    _add_tc,
    grid=(8,),
    in_specs=[
        pl.BlockSpec((1564, 128), lambda i: (i, 0)),
        pl.BlockSpec((1564, 128), lambda i: (i, 0)),
    ],
    out_specs=pl.BlockSpec((1564, 128), lambda i: (i, 0)),
    out_shape=jax.ShapeDtypeStruct((NP * 16 // 128, 128), f32),
)

_table0 = _make_table(NP // 8, 128, 256, 16)
_table12 = _make_table(NP // 4, 128, 128, 16)
_mid32 = _make_mid(128)
_mid16 = _make_mid(64)


def _prep_first_layer(W, b, fin, copies):
    wa = W[:fin]
    wb = W[fin:]
    eye = jnp.eye(copies, dtype=f32)
    wd_bd = jnp.kron(eye, wa - wb)
    wb_bd = jnp.kron(eye, wb)
    b_t = jnp.tile(b, copies)[None, :]
    return wd_bd, wb_bd, b_t


def _prep_mid(W1, b1, W2, b2):
    eye = jnp.eye(4, dtype=f32)
    return (jnp.kron(eye, W1), jnp.tile(b1, 4)[None, :],
            jnp.kron(eye, W2), jnp.tile(b2, 4)[None, :])


def kernel(x, pos, edge_index, batch,
           W0_0, b0_0, W0_1, b0_1, W0_2, b0_2,
           W1_0, b1_0, W1_1, b1_1, W1_2, b1_2,
           W2_0, b2_0, W2_1, b2_1, W2_2, b2_2):
    src = edge_index[0]
    dst = edge_index[1]
    pad_e = EP - E
    dst_g = jnp.pad(dst, (0, pad_e)).reshape(CH, 128)
    src_g = jnp.pad(src, (0, pad_e)).reshape(CH, 128)
    dst_s = jnp.pad(dst, (0, pad_e), constant_values=N).reshape(CH, 128)
    xp = jnp.pad(x, ((0, NP - N), (0, 0)))

    wd, wb, bt = _prep_first_layer(W0_0, b0_0, 16, 8)
    a_t, b_t = _table0(xp.reshape(NP // 8, 128), wd, wb, bt)
    h1 = _gather_call(a_t.reshape(NP, 32), b_t.reshape(NP, 32), dst_g, src_g)
    m1, bm1, m2, bm2 = _prep_mid(W0_1, b0_1, W0_2, b0_2)
    h3 = _mid32(h1.reshape(EP // 4, 128), m1, bm1, m2, bm2)
    h = _scatter_fs_call(h3.reshape(EP, 32), dst_s)

    wd, wb, bt = _prep_first_layer(W1_0, b1_0, 32, 4)
    a_t, b_t = _table12(h.reshape(NP // 4, 128), wd, wb, bt)
    h1 = _gather_call(a_t.reshape(NP, 32), b_t.reshape(NP, 32), dst_g, src_g)
    m1, bm1, m2, bm2 = _prep_mid(W1_1, b1_1, W1_2, b1_2)
    h3 = _mid32(h1.reshape(EP // 4, 128), m1, bm1, m2, bm2)
    h = _scatter_fs_call(h3.reshape(EP, 32), dst_s)

    wd, wb, bt = _prep_first_layer(W2_0, b2_0, 32, 4)
    a_t, b_t = _table12(h.reshape(NP // 4, 128), wd, wb, bt)
    h1 = _gather_call(a_t.reshape(NP, 32), b_t.reshape(NP, 32), dst_g, src_g)
    m1, bm1, m2, bm2 = _prep_mid(W2_1, b2_1, W2_2, b2_2)
    h3 = _mid16(h1.reshape(EP // 4, 128), m1, bm1, m2, bm2)
    acc0, acc1 = _scatter_es_call(h3.reshape(EP, 16), dst_s)
    out = _add_call(acc0.reshape(NP * 16 // 128, 128),
                    acc1.reshape(NP * 16 // 128, 128))
    return out.reshape(NP, 16)[:N]


# speedup vs baseline: 10.0516x; 10.0516x over previous
import functools

import jax
import jax.numpy as jnp
from jax import lax
from jax.experimental import pallas as pl
from jax.experimental.pallas import tpu as pltpu
from jax.experimental.pallas import tpu_sc as plsc

f32 = jnp.float32

N = 100000
NP = 100096
E = 1600000
EP = 1605632
CH = EP // 128
NW = 32
CPT = CH // NW
CPS = CH // 16
GK = 4
SK = 8
ROWS_G = GK * 128
ROWS_S = SK * 128
SLAB = NP // 16
ZR = 782

_mesh = plsc.VectorSubcoreMesh(core_axis_name="c", subcore_axis_name="s")
_sc_params = pltpu.CompilerParams(use_tc_tiling_on_sc=False)


def _gather_body(a_hbm, b_hbm, dstg, srcg, h1,
                 idxd0, idxs0, idxd1, idxs1,
                 bufa0, bufb0, bufa1, bufb1, sem0, sem1):
    cid = lax.axis_index("c")
    sid = lax.axis_index("s")
    wid = sid * 2 + cid
    base = wid * CPT
    idxd = (idxd0, idxd1)
    idxs = (idxs0, idxs1)
    bufa = (bufa0, bufa1)
    bufb = (bufb0, bufb1)
    sem = (sem0, sem1)

    def fire(h, b):
        c0 = base + h * GK
        pltpu.sync_copy(dstg.at[pl.ds(c0, GK)], idxd[b])
        pltpu.sync_copy(srcg.at[pl.ds(c0, GK)], idxs[b])
        for j in range(GK):
            sl = pl.ds(j * 128, 128)
            pltpu.async_copy(a_hbm.at[idxd[b].at[j]], bufa[b].at[sl], sem[b])
            pltpu.async_copy(b_hbm.at[idxs[b].at[j]], bufb[b].at[sl], sem[b])

    def drain(b):
        for j in range(GK):
            sl = pl.ds(j * 128, 128)
            pltpu.make_async_copy(a_hbm.at[idxd[b].at[j]], bufa[b].at[sl], sem[b]).wait()
            pltpu.make_async_copy(b_hbm.at[idxs[b].at[j]], bufb[b].at[sl], sem[b]).wait()

    fire(0, 0)
    fire(1, 1)
    n_groups = CPT // GK

    def outer(g, carry):
        for b in range(2):
            h = g * 2 + b
            drain(b)

            def add_body(i, c):
                for r in range(4):
                    row = i * 4 + r
                    for half in range(2):
                        sl = pl.ds(half * 16, 16)
                        bufa[b][row, sl] = bufa[b][row, sl] + bufb[b][row, sl]
                return c

            lax.fori_loop(0, ROWS_G // 4, add_body, 0)
            c0 = base + h * GK
            pltpu.sync_copy(bufa[b], h1.at[pl.ds(c0 * 128, ROWS_G)])

            @pl.when(h + 2 < n_groups)
            def _():
                fire(h + 2, b)
        return carry

    lax.fori_loop(0, n_groups // 2, outer, 0)


_gather_call = functools.partial(
    pl.kernel,
    out_type=jax.ShapeDtypeStruct((EP, 32), f32),
    mesh=_mesh,
    scratch_types=[
        pltpu.VMEM((GK, 128), jnp.int32),
        pltpu.VMEM((GK, 128), jnp.int32),
        pltpu.VMEM((GK, 128), jnp.int32),
        pltpu.VMEM((GK, 128), jnp.int32),
        pltpu.VMEM((ROWS_G, 32), f32),
        pltpu.VMEM((ROWS_G, 32), f32),
        pltpu.VMEM((ROWS_G, 32), f32),
        pltpu.VMEM((ROWS_G, 32), f32),
        pltpu.SemaphoreType.DMA,
        pltpu.SemaphoreType.DMA,
    ],
    compiler_params=_sc_params,
)(_gather_body)


def _zero_acc(acc, zbuf, sid):
    def zrow(i, c):
        zbuf[i, pl.ds(0, 16)] = jnp.zeros((16,), f32)
        return c

    lax.fori_loop(0, ZR, zrow, 0)
    for t in range(8):
        pltpu.sync_copy(zbuf, acc.at[pl.ds(sid * SLAB + t * ZR, ZR)])


def _scatter_fs_body(h3, dsts, out, acc, idxv, rows, zbuf, sem):
    cid = lax.axis_index("c")
    sid = lax.axis_index("s")
    _zero_acc(acc, zbuf, sid)
    plsc.subcore_barrier()
    base = sid * CPS

    def grp(g, carry):
        c0 = base + g * SK
        pltpu.sync_copy(dsts.at[pl.ds(c0, SK)], idxv)
        pltpu.sync_copy(h3.at[pl.ds(c0 * 128, ROWS_S), pl.ds(cid * 16, 16)], rows)
        for j in range(SK):
            sl = pl.ds(j * 128, 128)
            pltpu.async_copy(rows.at[sl], acc.at[idxv.at[j]], sem, add=True)
        for j in range(SK):
            sl = pl.ds(j * 128, 128)
            pltpu.make_async_copy(rows.at[sl], acc.at[idxv.at[j]], sem).wait()
        return carry

    lax.fori_loop(0, CPS // SK, grp, 0)
    plsc.subcore_barrier()
    pltpu.sync_copy(acc.at[pl.ds(sid * SLAB, SLAB)],
                    out.at[pl.ds(sid * SLAB, SLAB), pl.ds(cid * 16, 16)])


_scatter_fs_call = functools.partial(
    pl.kernel,
    out_type=jax.ShapeDtypeStruct((NP, 32), f32),
    mesh=_mesh,
    scratch_types=[
        pltpu.VMEM_SHARED((NP, 16), f32),
        pltpu.VMEM((SK, 128), jnp.int32),
        pltpu.VMEM((ROWS_S, 16), f32),
        pltpu.VMEM((ZR, 16), f32),
        pltpu.SemaphoreType.DMA,
    ],
    compiler_params=_sc_params,
)(_scatter_fs_body)


def _scatter_es_body(h3, dsts, out0, out1, acc, idxv, rows, zbuf, sem):
    cid = lax.axis_index("c")
    sid = lax.axis_index("s")
    _zero_acc(acc, zbuf, sid)
    plsc.subcore_barrier()
    wid = sid * 2 + cid
    base = wid * CPT

    def grp(g, carry):
        c0 = base + g * SK
        pltpu.sync_copy(dsts.at[pl.ds(c0, SK)], idxv)
        pltpu.sync_copy(h3.at[pl.ds(c0 * 128, ROWS_S)], rows)
        for j in range(SK):
            sl = pl.ds(j * 128, 128)
            pltpu.async_copy(rows.at[sl], acc.at[idxv.at[j]], sem, add=True)
        for j in range(SK):
            sl = pl.ds(j * 128, 128)
            pltpu.make_async_copy(rows.at[sl], acc.at[idxv.at[j]], sem).wait()
        return carry

    lax.fori_loop(0, CPT // SK, grp, 0)
    plsc.subcore_barrier()
    slab = pl.ds(sid * SLAB, SLAB)

    @pl.when(cid == 0)
    def _():
        pltpu.sync_copy(acc.at[slab], out0.at[slab])

    @pl.when(cid == 1)
    def _():
        pltpu.sync_copy(acc.at[slab], out1.at[slab])


_scatter_es_call = functools.partial(
    pl.kernel,
    out_type=[jax.ShapeDtypeStruct((NP, 16), f32),
              jax.ShapeDtypeStruct((NP, 16), f32)],
    mesh=_mesh,
    scratch_types=[
        pltpu.VMEM_SHARED((NP, 16), f32),
        pltpu.VMEM((SK, 128), jnp.int32),
        pltpu.VMEM((ROWS_S, 16), f32),
        pltpu.VMEM((ZR, 16), f32),
        pltpu.SemaphoreType.DMA,
    ],
    compiler_params=_sc_params,
)(_scatter_es_body)


def _table_tc(x_ref, wd_ref, wb_ref, bd_ref, a_ref, b_ref):
    xv = x_ref[...]
    a_ref[...] = jnp.dot(xv, wd_ref[...], preferred_element_type=f32) + bd_ref[...]
    b_ref[...] = jnp.dot(xv, wb_ref[...], preferred_element_type=f32)


def _make_table(nrows, in_cols, out_cols, grid):
    blk = nrows // grid
    return pl.pallas_call(
        _table_tc,
        grid=(grid,),
        in_specs=[
            pl.BlockSpec((blk, in_cols), lambda i: (i, 0)),
            pl.BlockSpec((in_cols, out_cols), lambda i: (0, 0)),
            pl.BlockSpec((in_cols, out_cols), lambda i: (0, 0)),
            pl.BlockSpec((1, out_cols), lambda i: (0, 0)),
        ],
        out_specs=[
            pl.BlockSpec((blk, out_cols), lambda i: (i, 0)),
            pl.BlockSpec((blk, out_cols), lambda i: (i, 0)),
        ],
        out_shape=[
            jax.ShapeDtypeStruct((nrows, out_cols), f32),
            jax.ShapeDtypeStruct((nrows, out_cols), f32),
        ],
    )


def _mid_tc(h_ref, w1_ref, b1_ref, w2_ref, b2_ref, o_ref):
    t = jnp.maximum(h_ref[...], 0.0)
    t = jnp.maximum(jnp.dot(t, w1_ref[...], preferred_element_type=f32) + b1_ref[...], 0.0)
    o_ref[...] = jnp.dot(t, w2_ref[...], preferred_element_type=f32) + b2_ref[...]


def _make_mid(out_cols):
    nrows = EP // 4
    blk = 1024
    grid = nrows // blk
    return pl.pallas_call(
        _mid_tc,
        grid=(grid,),
        in_specs=[
            pl.BlockSpec((blk, 128), lambda i: (i, 0)),
            pl.BlockSpec((128, 128), lambda i: (0, 0)),
            pl.BlockSpec((1, 128), lambda i: (0, 0)),
            pl.BlockSpec((128, out_cols), lambda i: (0, 0)),
            pl.BlockSpec((1, out_cols), lambda i: (0, 0)),
        ],
        out_specs=pl.BlockSpec((blk, out_cols), lambda i: (i, 0)),
        out_shape=jax.ShapeDtypeStruct((nrows, out_cols), f32),
    )


def _add_tc(a_ref, b_ref, o_ref):
    o_ref[...] = a_ref[...] + b_ref[...]


_add_call = pl.pallas_call(
    _add_tc,
    grid=(4,),
    in_specs=[
        pl.BlockSpec((3128, 128), lambda i: (i, 0)),
        pl.BlockSpec((3128, 128), lambda i: (i, 0)),
    ],
    out_specs=pl.BlockSpec((3128, 128), lambda i: (i, 0)),
    out_shape=jax.ShapeDtypeStruct((NP * 16 // 128, 128), f32),
)

_table0 = _make_table(NP // 8, 128, 256, 4)
_table12 = _make_table(NP // 4, 128, 128, 4)
_mid32 = _make_mid(128)
_mid16 = _make_mid(64)


def _prep_first_layer(W, b, fin, copies):
    wa = W[:fin]
    wb = W[fin:]
    eye = jnp.eye(copies, dtype=f32)
    wd_bd = jnp.kron(eye, wa - wb)
    wb_bd = jnp.kron(eye, wb)
    b_t = jnp.tile(b, copies)[None, :]
    return wd_bd, wb_bd, b_t


def _prep_mid(W1, b1, W2, b2):
    eye = jnp.eye(4, dtype=f32)
    return (jnp.kron(eye, W1), jnp.tile(b1, 4)[None, :],
            jnp.kron(eye, W2), jnp.tile(b2, 4)[None, :])


def kernel(x, pos, edge_index, batch,
           W0_0, b0_0, W0_1, b0_1, W0_2, b0_2,
           W1_0, b1_0, W1_1, b1_1, W1_2, b1_2,
           W2_0, b2_0, W2_1, b2_1, W2_2, b2_2):
    src = edge_index[0]
    dst = edge_index[1]
    pad_e = EP - E
    dst_g = jnp.pad(dst, (0, pad_e)).reshape(CH, 128)
    src_g = jnp.pad(src, (0, pad_e)).reshape(CH, 128)
    dst_s = jnp.pad(dst, (0, pad_e), constant_values=N).reshape(CH, 128)
    xp = jnp.pad(x, ((0, NP - N), (0, 0)))

    wd, wb, bt = _prep_first_layer(W0_0, b0_0, 16, 8)
    a_t, b_t = _table0(xp.reshape(NP // 8, 128), wd, wb, bt)
    h1 = _gather_call(a_t.reshape(NP, 32), b_t.reshape(NP, 32), dst_g, src_g)
    m1, bm1, m2, bm2 = _prep_mid(W0_1, b0_1, W0_2, b0_2)
    h3 = _mid32(h1.reshape(EP // 4, 128), m1, bm1, m2, bm2)
    h = _scatter_fs_call(h3.reshape(EP, 32), dst_s)

    wd, wb, bt = _prep_first_layer(W1_0, b1_0, 32, 4)
    a_t, b_t = _table12(h.reshape(NP // 4, 128), wd, wb, bt)
    h1 = _gather_call(a_t.reshape(NP, 32), b_t.reshape(NP, 32), dst_g, src_g)
    m1, bm1, m2, bm2 = _prep_mid(W1_1, b1_1, W1_2, b1_2)
    h3 = _mid32(h1.reshape(EP // 4, 128), m1, bm1, m2, bm2)
    h = _scatter_fs_call(h3.reshape(EP, 32), dst_s)

    wd, wb, bt = _prep_first_layer(W2_0, b2_0, 32, 4)
    a_t, b_t = _table12(h.reshape(NP // 4, 128), wd, wb, bt)
    h1 = _gather_call(a_t.reshape(NP, 32), b_t.reshape(NP, 32), dst_g, src_g)
    m1, bm1, m2, bm2 = _prep_mid(W2_1, b2_1, W2_2, b2_2)
    h3 = _mid16(h1.reshape(EP // 4, 128), m1, bm1, m2, bm2)
    acc0, acc1 = _scatter_es_call(h3.reshape(EP, 16), dst_s)
    out = _add_call(acc0.reshape(NP * 16 // 128, 128),
                    acc1.reshape(NP * 16 // 128, 128))
    return out.reshape(NP, 16)[:N]
